# Initial kernel scaffold; baseline (speedup 1.0000x reference)
import jax
import jax.numpy as jnp
from jax.experimental import pallas as pl


def kernel(x, W0, W1):
    raise NotImplementedError("write your pallas kernel here")


# speedup vs baseline: 8.9568x; 8.9568x over previous
import jax
import jax.numpy as jnp
from jax.experimental import pallas as pl

N = 100000
HALF = 64
BLOCK = 2000
NB = N // BLOCK


def _body(idx0_ref, idx1_ref, w0_ref, w1_ref, out_ref):
    i0 = jnp.clip(idx0_ref[0, 0, :], 0, w0_ref.shape[0] - 1)
    i1 = jnp.clip(idx1_ref[0, 0, :], 0, w1_ref.shape[0] - 1)
    k0 = jax.lax.broadcasted_iota(jnp.int32, (BLOCK, w0_ref.shape[0]), 1)
    k1 = jax.lax.broadcasted_iota(jnp.int32, (BLOCK, w1_ref.shape[0]), 1)
    oh0 = (i0[:, None] == k0).astype(jnp.float32)
    oh1 = (i1[:, None] == k1).astype(jnp.float32)
    out_ref[:, :HALF] = jnp.dot(oh0, w0_ref[...],
                                preferred_element_type=jnp.float32)
    out_ref[:, HALF:] = jnp.dot(oh1, w1_ref[...],
                                preferred_element_type=jnp.float32)


def kernel(x, W0, W1):
    xi = x.astype(jnp.int32)
    idx0 = xi[:, 0].reshape(NB, 1, BLOCK)
    idx1 = xi[:, 1].reshape(NB, 1, BLOCK)
    return pl.pallas_call(
        _body,
        grid=(NB,),
        in_specs=[
            pl.BlockSpec((1, 1, BLOCK), lambda i: (i, 0, 0)),
            pl.BlockSpec((1, 1, BLOCK), lambda i: (i, 0, 0)),
            pl.BlockSpec(W0.shape, lambda i: (0, 0)),
            pl.BlockSpec(W1.shape, lambda i: (0, 0)),
        ],
        out_specs=pl.BlockSpec((BLOCK, 2 * HALF), lambda i: (i, 0)),
        out_shape=jax.ShapeDtypeStruct((N, 2 * HALF), jnp.float32),
    )(idx0, idx1, W0, W1)


# speedup vs baseline: 9.8451x; 1.0992x over previous
import jax
import jax.numpy as jnp
from jax.experimental import pallas as pl

N = 100000
HALF = 64
BLOCK = 10000
NB = N // BLOCK


def _body(idx0_ref, idx1_ref, w0_ref, w1_ref, out_ref):
    i0 = jnp.clip(idx0_ref[0, 0, :], 0, w0_ref.shape[0] - 1)
    i1 = jnp.clip(idx1_ref[0, 0, :], 0, w1_ref.shape[0] - 1)
    k0 = jax.lax.broadcasted_iota(jnp.int32, (BLOCK, w0_ref.shape[0]), 1)
    k1 = jax.lax.broadcasted_iota(jnp.int32, (BLOCK, w1_ref.shape[0]), 1)
    oh0 = (i0[:, None] == k0).astype(jnp.float32)
    oh1 = (i1[:, None] == k1).astype(jnp.float32)
    out_ref[:, :HALF] = jnp.dot(oh0, w0_ref[...],
                                preferred_element_type=jnp.float32)
    out_ref[:, HALF:] = jnp.dot(oh1, w1_ref[...],
                                preferred_element_type=jnp.float32)


def kernel(x, W0, W1):
    xi = x.astype(jnp.int32)
    idx0 = xi[:, 0].reshape(NB, 1, BLOCK)
    idx1 = xi[:, 1].reshape(NB, 1, BLOCK)
    return pl.pallas_call(
        _body,
        grid=(NB,),
        in_specs=[
            pl.BlockSpec((1, 1, BLOCK), lambda i: (i, 0, 0)),
            pl.BlockSpec((1, 1, BLOCK), lambda i: (i, 0, 0)),
            pl.BlockSpec(W0.shape, lambda i: (0, 0)),
            pl.BlockSpec(W1.shape, lambda i: (0, 0)),
        ],
        out_specs=pl.BlockSpec((BLOCK, 2 * HALF), lambda i: (i, 0)),
        out_shape=jax.ShapeDtypeStruct((N, 2 * HALF), jnp.float32),
    )(idx0, idx1, W0, W1)
